# Initial kernel scaffold; baseline (speedup 1.0000x reference)
import jax
import jax.numpy as jnp
from jax.experimental import pallas as pl


def kernel(enc_se0_w, enc_se0_b, enc_se1_w, enc_se1_b, enc_se2_w, enc_se2_b, enc_se3_w, enc_se3_b, enc_sc01_w, enc_sc01_b, enc_sc12_w, enc_sc12_b, enc_t001_w, enc_t001_b, enc_t012_w, enc_t012_b, enc_t101_w, enc_t101_b, enc_t112_w, enc_t112_b, enc_t201_w, enc_t201_b, enc_t212_w, enc_t212_b, emb21_w, emb21_b, emb11_w, emb11_b, emb10_w, emb10_b, fusion_w, fusion_b, memory_w0, memory_w1, memory_w2, memory_x):
    raise NotImplementedError("write your pallas kernel here")


# speedup vs baseline: 2.6566x; 2.6566x over previous
import functools

import jax
import jax.numpy as jnp
from jax.experimental import pallas as pl
from jax.experimental.pallas import tpu as pltpu

_VMEM = 64 * 1024 * 1024


def _act(y, kind):
    if kind == "relu":
        return jnp.maximum(y, 0.0)
    if kind == "elu":
        return jnp.where(y > 0, y, jnp.exp(jnp.minimum(y, 0.0)) - 1.0)
    return y


def _conv3_body(x_ref, w_ref, b_ref, o_ref, *, taps, act, oh, ow, phased):
    c = x_ref.shape[-1]
    n = o_ref.shape[-1]
    acc = jnp.zeros((oh * ow, n), jnp.float32)
    for (a, pb, i1, j1, kh, kw) in taps:
        if phased:
            xs = x_ref[0, a, pb, i1:i1 + oh, j1:j1 + ow, :]
        else:
            xs = x_ref[0, i1:i1 + oh, j1:j1 + ow, :]
        xs = xs.astype(jnp.float32).reshape(oh * ow, c).astype(jnp.bfloat16)
        acc = acc + jnp.dot(xs, w_ref[kh, kw],
                            preferred_element_type=jnp.float32)
    o_ref[...] = _act(acc + b_ref[...], act).astype(o_ref.dtype)


def _conv3x3(x, w, b, stride, act):
    N, H, W, C = x.shape
    OC = w.shape[0]
    OH, OW = H // stride, W // stride
    xb = jnp.pad(x.astype(jnp.bfloat16), ((0, 0), (1, 1), (1, 1), (0, 0)))
    if stride == 1:
        taps = tuple((0, 0, kh, kw, kh, kw) for kh in range(3) for kw in range(3))
        xin = xb
        in_spec = pl.BlockSpec((1, H + 2, W + 2, C), lambda i: (i, 0, 0, 0))
        phased = False
    else:
        Hh, Wh = (H + 2) // 2, (W + 2) // 2
        xin = xb.reshape(N, Hh, 2, Wh, 2, C).transpose(0, 2, 4, 1, 3, 5)
        taps = tuple((kh % 2, kw % 2, kh // 2, kw // 2, kh, kw)
                     for kh in range(3) for kw in range(3))
        in_spec = pl.BlockSpec((1, 2, 2, Hh, Wh, C),
                               lambda i: (i, 0, 0, 0, 0, 0))
        phased = True
    wt = w.transpose(2, 3, 1, 0).astype(jnp.bfloat16)
    bb = b.reshape(1, OC).astype(jnp.float32)
    out = pl.pallas_call(
        functools.partial(_conv3_body, taps=taps, act=act, oh=OH, ow=OW,
                          phased=phased),
        out_shape=jax.ShapeDtypeStruct((N * OH * OW, OC), jnp.bfloat16),
        grid=(N,),
        in_specs=[
            in_spec,
            pl.BlockSpec((3, 3, C, OC), lambda i: (0, 0, 0, 0)),
            pl.BlockSpec((1, OC), lambda i: (0, 0)),
        ],
        out_specs=pl.BlockSpec((OH * OW, OC), lambda i: (i, 0)),
        compiler_params=pltpu.CompilerParams(
            dimension_semantics=("parallel",), vmem_limit_bytes=_VMEM),
    )(xin, wt, bb)
    return out.reshape(N, OH, OW, OC)


def _mm_body(x_ref, w_ref, b_ref, o_ref, *, act):
    y = jnp.dot(x_ref[...], w_ref[...], preferred_element_type=jnp.float32)
    o_ref[...] = _act(y + b_ref[...], act).astype(o_ref.dtype)


def _mm(x, w, b, act, grid_m):
    M, K = x.shape
    N = w.shape[1]
    tm = M // grid_m
    return pl.pallas_call(
        functools.partial(_mm_body, act=act),
        out_shape=jax.ShapeDtypeStruct((M, N), jnp.bfloat16),
        grid=(grid_m,),
        in_specs=[
            pl.BlockSpec((tm, K), lambda i: (i, 0)),
            pl.BlockSpec((K, N), lambda i: (0, 0)),
            pl.BlockSpec((1, N), lambda i: (0, 0)),
        ],
        out_specs=pl.BlockSpec((tm, N), lambda i: (i, 0)),
        compiler_params=pltpu.CompilerParams(
            dimension_semantics=("parallel",), vmem_limit_bytes=_VMEM),
    )(x.astype(jnp.bfloat16), w.astype(jnp.bfloat16),
      b.reshape(1, N).astype(jnp.float32))


def _read(q, mem_n, mem_raw):
    qn = q * jax.lax.rsqrt(
        jnp.maximum(jnp.sum(q * q, axis=1, keepdims=True), 1e-24))
    s = jax.lax.dot_general(qn.astype(jnp.bfloat16), mem_n,
                            (((1,), (1,)), ((), ())),
                            preferred_element_type=jnp.float32)
    s = jnp.exp(s - jnp.max(s, axis=1, keepdims=True))
    p = s / jnp.sum(s, axis=1, keepdims=True)
    return jnp.dot(p.astype(jnp.bfloat16), mem_raw,
                   preferred_element_type=jnp.float32)


def _patch4(x4d):
    b, h, w, C = x4d.shape
    zrow = jnp.zeros((b, 1, w, C), x4d.dtype)
    xp = jnp.concatenate([x4d, zrow], axis=1)
    zcol = jnp.zeros((b, h + 1, 1, C), x4d.dtype)
    xp = jnp.concatenate([xp, zcol], axis=2)
    cols = jnp.concatenate(
        [xp[:, di:di + h, dj:dj + w, :] for di in (0, 1) for dj in (0, 1)],
        axis=-1)
    return cols.reshape(b * h * w, 4 * C)


def _mlp2(x, w1, b1, w2, b2):
    h = jnp.dot(x, w1, preferred_element_type=jnp.float32)
    h = jnp.maximum(h + b1, 0.0).astype(jnp.bfloat16)
    y = jnp.dot(h, w2, preferred_element_type=jnp.float32)
    return jnp.maximum(y + b2, 0.0)


def _head2_body(x_ref, w1_ref, b1_ref, w2_ref, b2_ref, mn_ref, mr_ref,
                wc_ref, bc_ref, o_ref, *, b, h, w):
    xr = x_ref[...].reshape(b * h * w, x_ref.shape[-1])
    r2 = _mlp2(xr, w1_ref[...], b1_ref[...], w2_ref[...], b2_ref[...])
    mf = _read(r2, mn_ref[...], mr_ref[...])
    C = mf.shape[-1]
    cols = _patch4(mf.astype(jnp.bfloat16).reshape(b, h, w, C))
    y = jnp.dot(cols, wc_ref[...], preferred_element_type=jnp.float32)
    o_ref[...] = jnp.maximum(y + bc_ref[...], 0.0).astype(o_ref.dtype)


def _head1_body(x_ref, mf2_ref, w1_ref, b1_ref, w2_ref, b2_ref, mn_ref,
                mr_ref, w10_ref, b10_ref, w11_ref, b11_ref, o10_ref, o11_ref,
                *, b, h, w):
    xr = x_ref[...].reshape(b * h * w, x_ref.shape[-1])
    r1 = _mlp2(xr, w1_ref[...], b1_ref[...], w2_ref[...], b2_ref[...])
    C = mf2_ref.shape[-1]
    mf2 = mf2_ref[...].reshape(b * h * w, C)
    mq = (r1.astype(jnp.bfloat16) + mf2).astype(jnp.float32)
    mf = _read(mq, mn_ref[...], mr_ref[...])
    cols = _patch4(mf.astype(jnp.bfloat16).reshape(b, h, w, C))
    y = jnp.dot(cols, w10_ref[...], preferred_element_type=jnp.float32)
    o10_ref[...] = jnp.maximum(y + b10_ref[...], 0.0).astype(o10_ref.dtype)
    cols2 = _patch4(mf2_ref[...])
    y2 = jnp.dot(cols2, w11_ref[...], preferred_element_type=jnp.float32)
    o11_ref[...] = jnp.maximum(y2 + b11_ref[...], 0.0).astype(o11_ref.dtype)


def _head0_body(x_ref, mf1_ref, up2_ref, w1_ref, b1_ref, w2_ref, b2_ref,
                mn_ref, mr_ref, wf0_ref, wf1_ref, wf2_ref, bf_ref, o_ref):
    r0 = _mlp2(x_ref[...], w1_ref[...], b1_ref[...], w2_ref[...], b2_ref[...])
    mf1 = mf1_ref[...]
    mq = (r0.astype(jnp.bfloat16) + mf1).astype(jnp.float32)
    mf0 = _read(mq, mn_ref[...], mr_ref[...])
    y = (jnp.dot(mf0.astype(jnp.bfloat16), wf0_ref[...],
                 preferred_element_type=jnp.float32)
         + jnp.dot(mf1, wf1_ref[...], preferred_element_type=jnp.float32)
         + jnp.dot(up2_ref[...], wf2_ref[...],
                   preferred_element_type=jnp.float32))
    o_ref[...] = jnp.maximum(y + bf_ref[...], 0.0)


def _group(x, B, T):
    Bt, H, W, C = x.shape
    return (x.reshape(B, T, H, W, C).transpose(0, 2, 3, 1, 4)
            .reshape(B, H, W, T * C))


def _norm_rows(m):
    mf = m.astype(jnp.float32)
    return (mf * jax.lax.rsqrt(
        jnp.maximum(jnp.sum(mf * mf, axis=1, keepdims=True), 1e-24))
            ).astype(jnp.bfloat16)


def _convT_weight(w):
    IC, OC = w.shape[0], w.shape[1]
    zeros = jnp.zeros((IC, OC), w.dtype)
    blocks = []
    for di in (0, 1):
        for dj in (0, 1):
            taps = []
            for ph in (0, 1):
                for pw in (0, 1):
                    kh = ph - 2 * di + 1
                    kw = pw - 2 * dj + 1
                    taps.append(w[:, :, kh, kw]
                                if (0 <= kh < 3 and 0 <= kw < 3) else zeros)
            blocks.append(jnp.stack(taps, axis=1).reshape(IC, 4 * OC))
    return jnp.concatenate(blocks, axis=0).astype(jnp.bfloat16)


def _shuffle(y, B, H, W, OC):
    y = y.reshape(B, H, W, 2, 2, OC).transpose(0, 1, 3, 2, 4, 5)
    return y.reshape(B, 2 * H, 2 * W, OC)


def _bspec(shape):
    n = len(shape)
    return pl.BlockSpec(shape, lambda i, _n=n: (0,) * _n)


def kernel(enc_se0_w, enc_se0_b, enc_se1_w, enc_se1_b, enc_se2_w, enc_se2_b,
           enc_se3_w, enc_se3_b, enc_sc01_w, enc_sc01_b, enc_sc12_w,
           enc_sc12_b, enc_t001_w, enc_t001_b, enc_t012_w, enc_t012_b,
           enc_t101_w, enc_t101_b, enc_t112_w, enc_t112_b, enc_t201_w,
           enc_t201_b, enc_t212_w, enc_t212_b, emb21_w, emb21_b, emb11_w,
           emb11_b, emb10_w, emb10_b, fusion_w, fusion_b, memory_w0,
           memory_w1, memory_w2, memory_x):
    B, L = memory_x.shape[0], memory_x.shape[1]
    T = L - 1
    H = memory_x.shape[3]
    N = B * T

    x = memory_x[:, :, 0]
    d = (x[:, 1:] - x[:, :-1]).reshape(N, H, H)
    dp = jnp.pad(d.astype(jnp.bfloat16), ((0, 0), (1, 1), (1, 1)))
    cols = jnp.stack([dp[:, kh:kh + H:2, kw:kw + H:2]
                      for kh in range(3) for kw in range(3)], axis=-1)
    h0 = H // 2
    w0 = enc_se0_w.reshape(64, 9).T
    y = _mm(cols.reshape(N * h0 * h0, 9), w0, enc_se0_b, "elu", 16)
    x = y.reshape(N, h0, h0, 64)

    x = _conv3x3(x, enc_se1_w, enc_se1_b, 1, "elu")
    x = _conv3x3(x, enc_se2_w, enc_se2_b, 2, "elu")
    x0 = _conv3x3(x, enc_se3_w, enc_se3_b, 1, "elu")
    x1 = _conv3x3(x0, enc_sc01_w, enc_sc01_b, 2, "relu")
    x2 = _conv3x3(x1, enc_sc12_w, enc_sc12_b, 2, "relu")

    x0g = _group(x0, B, T)
    x1g = _group(x1, B, T)
    x2g = _group(x2, B, T)

    def tw(w):
        return w.reshape(w.shape[0], w.shape[1]).T.astype(jnp.bfloat16)

    def bias(bv):
        return bv.reshape(1, -1).astype(jnp.float32)

    mn2, mr2 = _norm_rows(memory_w2), memory_w2.astype(jnp.bfloat16)
    mn1, mr1 = _norm_rows(memory_w1), memory_w1.astype(jnp.bfloat16)
    mn0, mr0 = _norm_rows(memory_w0), memory_w0.astype(jnp.bfloat16)

    bh = B // 2
    wc21 = _convT_weight(emb21_w)
    y2 = pl.pallas_call(
        functools.partial(_head2_body, b=bh, h=8, w=8),
        out_shape=jax.ShapeDtypeStruct((B * 64, 1024), jnp.bfloat16),
        grid=(2,),
        in_specs=[
            pl.BlockSpec((bh, 8, 8, 2048), lambda i: (i, 0, 0, 0)),
            _bspec((2048, 1024)), _bspec((1, 1024)),
            _bspec((1024, 512)), _bspec((1, 512)),
            _bspec((256, 512)), _bspec((256, 512)),
            _bspec((2048, 1024)), _bspec((1, 1024)),
        ],
        out_specs=pl.BlockSpec((bh * 64, 1024), lambda i: (i, 0)),
        compiler_params=pltpu.CompilerParams(
            dimension_semantics=("parallel",), vmem_limit_bytes=_VMEM),
    )(x2g.astype(jnp.bfloat16), tw(enc_t201_w), bias(enc_t201_b),
      tw(enc_t212_w), bias(enc_t212_b), mn2, mr2,
      wc21, bias(jnp.tile(emb21_b, 4)))
    mf2 = _shuffle(y2, B, 8, 8, 256)

    wc10 = _convT_weight(emb10_w)
    wc11 = _convT_weight(emb11_w)
    y10, y11 = pl.pallas_call(
        functools.partial(_head1_body, b=bh, h=16, w=16),
        out_shape=(jax.ShapeDtypeStruct((B * 256, 512), jnp.bfloat16),
                   jax.ShapeDtypeStruct((B * 256, 512), jnp.bfloat16)),
        grid=(2,),
        in_specs=[
            pl.BlockSpec((bh, 16, 16, 1024), lambda i: (i, 0, 0, 0)),
            pl.BlockSpec((bh, 16, 16, 256), lambda i: (i, 0, 0, 0)),
            _bspec((1024, 512)), _bspec((1, 512)),
            _bspec((512, 256)), _bspec((1, 256)),
            _bspec((512, 256)), _bspec((512, 256)),
            _bspec((1024, 512)), _bspec((1, 512)),
            _bspec((1024, 512)), _bspec((1, 512)),
        ],
        out_specs=(pl.BlockSpec((bh * 256, 512), lambda i: (i, 0)),
                   pl.BlockSpec((bh * 256, 512), lambda i: (i, 0))),
        compiler_params=pltpu.CompilerParams(
            dimension_semantics=("parallel",), vmem_limit_bytes=_VMEM),
    )(x1g.astype(jnp.bfloat16), mf2, tw(enc_t101_w), bias(enc_t101_b),
      tw(enc_t112_w), bias(enc_t112_b), mn1, mr1,
      wc10, bias(jnp.tile(emb10_b, 4)), wc11, bias(jnp.tile(emb11_b, 4)))
    mf1 = _shuffle(y10, B, 16, 16, 128).reshape(B * 1024, 128)
    up2 = _shuffle(y11, B, 16, 16, 128).reshape(B * 1024, 128)

    wf = fusion_w.reshape(384, 128).astype(jnp.bfloat16)
    yf = pl.pallas_call(
        _head0_body,
        out_shape=jax.ShapeDtypeStruct((B * 1024, 128), jnp.float32),
        grid=(4,),
        in_specs=[
            pl.BlockSpec((B * 256, 512), lambda i: (i, 0)),
            pl.BlockSpec((B * 256, 128), lambda i: (i, 0)),
            pl.BlockSpec((B * 256, 128), lambda i: (i, 0)),
            _bspec((512, 256)), _bspec((1, 256)),
            _bspec((256, 128)), _bspec((1, 128)),
            _bspec((1024, 128)), _bspec((1024, 128)),
            _bspec((128, 128)), _bspec((128, 128)), _bspec((128, 128)),
            _bspec((1, 128)),
        ],
        out_specs=pl.BlockSpec((B * 256, 128), lambda i: (i, 0)),
        compiler_params=pltpu.CompilerParams(
            dimension_semantics=("parallel",), vmem_limit_bytes=_VMEM),
    )(x0g.reshape(B * 1024, 512).astype(jnp.bfloat16), mf1, up2,
      tw(enc_t001_w), bias(enc_t001_b), tw(enc_t012_w), bias(enc_t012_b),
      mn0, mr0, wf[:128], wf[128:256], wf[256:], bias(fusion_b))
    return yf.reshape(B, 32, 32, 128).transpose(0, 3, 1, 2)


# speedup vs baseline: 2.9622x; 1.1150x over previous
import functools

import jax
import jax.numpy as jnp
from jax.experimental import pallas as pl
from jax.experimental.pallas import tpu as pltpu

_VMEM = 64 * 1024 * 1024


def _act(y, kind):
    if kind == "relu":
        return jnp.maximum(y, 0.0)
    if kind == "elu":
        return jnp.where(y > 0, y, jnp.exp(jnp.minimum(y, 0.0)) - 1.0)
    return y


def _dotT(x, w):
    return jax.lax.dot_general(x, w, (((1,), (1,)), ((), ())),
                               preferred_element_type=jnp.float32)


def _pad_hw(y3):
    h, w, c = y3.shape
    zr = jnp.zeros((1, w, c), y3.dtype)
    y3 = jnp.concatenate([zr, y3, zr], axis=0)
    zc = jnp.zeros((h + 2, 1, c), y3.dtype)
    return jnp.concatenate([zc, y3, zc], axis=1)


def _phase_split(yp):
    H, W, c = yp.shape
    return (yp.reshape(H // 2, 2, W // 2, 2, c)
            .transpose(1, 3, 0, 2, 4))


def _conv3_body(x_ref, w_ref, b_ref, *o_refs, taps, act, oh, ow,
                phased_in, out_modes):
    c = x_ref.shape[-1]
    n = w_ref.shape[-2]
    acc = jnp.zeros((oh * ow, n), jnp.float32)
    for (a, pb, i1, j1, kh, kw) in taps:
        if phased_in:
            xs = x_ref[0, a, pb, i1:i1 + oh, j1:j1 + ow, :]
        else:
            xs = x_ref[0, i1:i1 + oh, j1:j1 + ow, :]
        xs = xs.astype(jnp.float32).reshape(oh * ow, c).astype(jnp.bfloat16)
        acc = acc + _dotT(xs, w_ref[kh, kw])
    y = _act(acc + b_ref[...], act).astype(jnp.bfloat16)
    for o_ref, mode in zip(o_refs, out_modes):
        if mode == "rows":
            o_ref[...] = y
        else:
            yp = _pad_hw(y.reshape(oh, ow, n))
            if mode == "pad":
                o_ref[...] = yp[None]
            else:
                o_ref[...] = _phase_split(yp)[None]


def _conv3x3(x, xin_kind, w, b, stride, act, out_modes):
    N = x.shape[0]
    C = x.shape[-1]
    OC = w.shape[0]
    if xin_kind == "pad":
        H = x.shape[1] - 2
        assert stride == 1
        OH = OW = H
        taps = tuple((0, 0, kh, kw, kh, kw)
                     for kh in range(3) for kw in range(3))
        in_spec = pl.BlockSpec((1,) + x.shape[1:], lambda i: (i, 0, 0, 0))
        phased_in = False
    else:
        H = 2 * x.shape[3] - 2
        assert stride == 2
        OH = OW = H // 2
        taps = tuple((kh % 2, kw % 2, kh // 2, kw // 2, kh, kw)
                     for kh in range(3) for kw in range(3))
        in_spec = pl.BlockSpec((1,) + x.shape[1:],
                               lambda i: (i, 0, 0, 0, 0, 0))
        phased_in = True

    out_shapes = []
    out_specs = []
    for mode in out_modes:
        if mode == "rows":
            out_shapes.append(
                jax.ShapeDtypeStruct((N * OH * OW, OC), jnp.bfloat16))
            out_specs.append(pl.BlockSpec((OH * OW, OC), lambda i: (i, 0)))
        elif mode == "pad":
            out_shapes.append(
                jax.ShapeDtypeStruct((N, OH + 2, OW + 2, OC), jnp.bfloat16))
            out_specs.append(
                pl.BlockSpec((1, OH + 2, OW + 2, OC),
                             lambda i: (i, 0, 0, 0)))
        else:
            hh = (OH + 2) // 2
            out_shapes.append(
                jax.ShapeDtypeStruct((N, 2, 2, hh, hh, OC), jnp.bfloat16))
            out_specs.append(
                pl.BlockSpec((1, 2, 2, hh, hh, OC),
                             lambda i: (i, 0, 0, 0, 0, 0)))

    wt = w.reshape(OC, C, 9).transpose(2, 0, 1).reshape(3, 3, OC, C)
    outs = pl.pallas_call(
        functools.partial(_conv3_body, taps=taps, act=act, oh=OH, ow=OW,
                          phased_in=phased_in, out_modes=tuple(out_modes)),
        out_shape=tuple(out_shapes),
        grid=(N,),
        in_specs=[
            in_spec,
            pl.BlockSpec((3, 3, OC, C), lambda i: (0, 0, 0, 0)),
            pl.BlockSpec((1, OC), lambda i: (0, 0)),
        ],
        out_specs=tuple(out_specs),
        compiler_params=pltpu.CompilerParams(
            dimension_semantics=("parallel",), vmem_limit_bytes=_VMEM),
    )(x, wt.astype(jnp.bfloat16), b.reshape(1, OC).astype(jnp.float32))
    return outs


def _se0_body(c_ref, w_ref, b_ref, o_ref, *, oh):
    y = jnp.dot(c_ref[...], w_ref[...], preferred_element_type=jnp.float32)
    y = _act(y + b_ref[...], "elu").astype(jnp.bfloat16)
    o_ref[...] = _pad_hw(y.reshape(oh, oh, 64))[None]


def _read(q, mem_n, mem_raw):
    qn = q * jax.lax.rsqrt(
        jnp.maximum(jnp.sum(q * q, axis=1, keepdims=True), 1e-24))
    s = _dotT(qn.astype(jnp.bfloat16), mem_n)
    s = jnp.exp(s - jnp.max(s, axis=1, keepdims=True))
    p = s / jnp.sum(s, axis=1, keepdims=True)
    return jnp.dot(p.astype(jnp.bfloat16), mem_raw,
                   preferred_element_type=jnp.float32)


def _patch4(x4d):
    b, h, w, C = x4d.shape
    xp = jnp.concatenate([x4d, jnp.zeros((b, 1, w, C), x4d.dtype)], axis=1)
    xp = jnp.concatenate([xp, jnp.zeros((b, h + 1, 1, C), x4d.dtype)], axis=2)
    cols = jnp.concatenate(
        [xp[:, di:di + h, dj:dj + w, :] for di in (0, 1) for dj in (0, 1)],
        axis=-1)
    return cols.reshape(b * h * w, 4 * C)


def _shuffle_rows(y, b, h, w, oc):
    return (y.reshape(b, h, w, 2, 2, oc).transpose(0, 1, 3, 2, 4, 5)
            .reshape(b, 2 * h, 2 * w, oc))


def _tile4(bias_ref):
    bv = bias_ref[...]
    return jnp.concatenate([bv, bv, bv, bv], axis=1)


def _finish_mlp(acc, b1_ref, w2_ref, b2_ref):
    h = jnp.maximum(acc + b1_ref[...], 0.0).astype(jnp.bfloat16)
    return jnp.maximum(_dotT(h, w2_ref[...]) + b2_ref[...], 0.0)


def _head2_body(x_ref, w1_ref, b1_ref, w2_ref, b2_ref, mn_ref, mr_ref,
                wc_ref, bc_ref, o_ref, acc_ref, *, bh, T):
    t = pl.program_id(1)

    @pl.when(t == 0)
    def _():
        acc_ref[...] = jnp.zeros_like(acc_ref)

    xr = x_ref[...].reshape(bh * 64, 512)
    acc_ref[...] += _dotT(xr, w1_ref[...])

    @pl.when(t == T - 1)
    def _():
        r2 = _finish_mlp(acc_ref[...], b1_ref, w2_ref, b2_ref)
        mf = _read(r2, mn_ref[...], mr_ref[...])
        cols = _patch4(mf.astype(jnp.bfloat16).reshape(bh, 8, 8, 512))
        y = jnp.dot(cols, wc_ref[...], preferred_element_type=jnp.float32)
        o_ref[...] = jnp.maximum(y + _tile4(bc_ref), 0.0).astype(o_ref.dtype)


def _head1_body(x_ref, y2_ref, w1_ref, b1_ref, w2_ref, b2_ref, mn_ref,
                mr_ref, w10_ref, b10_ref, w11_ref, b11_ref,
                o10_ref, o11_ref, acc_ref, *, bh, T):
    t = pl.program_id(1)

    @pl.when(t == 0)
    def _():
        acc_ref[...] = jnp.zeros_like(acc_ref)

    xr = x_ref[...].reshape(bh * 256, 256)
    acc_ref[...] += _dotT(xr, w1_ref[...])

    @pl.when(t == T - 1)
    def _():
        r1 = _finish_mlp(acc_ref[...], b1_ref, w2_ref, b2_ref)
        mf2 = _shuffle_rows(y2_ref[...], bh, 8, 8, 256)
        mq = (r1.astype(jnp.bfloat16)
              + mf2.reshape(bh * 256, 256)).astype(jnp.float32)
        mf = _read(mq, mn_ref[...], mr_ref[...])
        cols = _patch4(mf.astype(jnp.bfloat16).reshape(bh, 16, 16, 256))
        y = jnp.dot(cols, w10_ref[...], preferred_element_type=jnp.float32)
        o10_ref[...] = jnp.maximum(y + _tile4(b10_ref), 0.0
                                   ).astype(o10_ref.dtype)
        cols2 = _patch4(mf2)
        y2 = jnp.dot(cols2, w11_ref[...], preferred_element_type=jnp.float32)
        o11_ref[...] = jnp.maximum(y2 + _tile4(b11_ref), 0.0
                                   ).astype(o11_ref.dtype)


def _head0_body(x_ref, y10_ref, y11_ref, w1_ref, b1_ref, w2_ref, b2_ref,
                mn_ref, mr_ref, wf_ref, bf_ref, o_ref, acc_ref, *, bh, T):
    t = pl.program_id(1)

    @pl.when(t == 0)
    def _():
        acc_ref[...] = jnp.zeros_like(acc_ref)

    xr = x_ref[...].reshape(bh * 1024, 128)
    acc_ref[...] += _dotT(xr, w1_ref[...])

    @pl.when(t == T - 1)
    def _():
        r0 = _finish_mlp(acc_ref[...], b1_ref, w2_ref, b2_ref)
        mf1 = _shuffle_rows(y10_ref[...].reshape(bh * 256, 512),
                            bh, 16, 16, 128).reshape(bh * 1024, 128)
        up2 = _shuffle_rows(y11_ref[...].reshape(bh * 256, 512),
                            bh, 16, 16, 128).reshape(bh * 1024, 128)
        mq = (r0.astype(jnp.bfloat16) + mf1).astype(jnp.float32)
        mf0 = _read(mq, mn_ref[...], mr_ref[...])
        wf = wf_ref[...]
        f32 = jnp.float32
        y = (jnp.dot(mf0.astype(jnp.bfloat16), wf[:128],
                     preferred_element_type=f32)
             + jnp.dot(mf1, wf[128:256], preferred_element_type=f32)
             + jnp.dot(up2, wf[256:], preferred_element_type=f32))
        y = jnp.maximum(y + bf_ref[...], 0.0)
        yt = jnp.transpose(y.reshape(bh, 1024, 128), (0, 2, 1))
        o_ref[...] = yt.reshape(bh, 128, 32, 32)


def _norm_rows(m):
    mf = m.astype(jnp.float32)
    return (mf * jax.lax.rsqrt(
        jnp.maximum(jnp.sum(mf * mf, axis=1, keepdims=True), 1e-24))
            ).astype(jnp.bfloat16)


def _convT_weight(w):
    IC, OC = w.shape[0], w.shape[1]
    wp = jnp.pad(w, ((0, 0), (0, 0), (1, 1), (1, 1)))
    blocks = jnp.stack(
        [wp[:, :, 2 - 2 * di:4 - 2 * di, 2 - 2 * dj:4 - 2 * dj]
         for di in (0, 1) for dj in (0, 1)], axis=0)
    wc = blocks.transpose(0, 1, 3, 4, 2).reshape(4 * IC, 4 * OC)
    return wc.astype(jnp.bfloat16)


def _bspec(shape):
    n = len(shape)
    return pl.BlockSpec(shape, lambda i, t, _n=n: (0,) * _n)


def kernel(enc_se0_w, enc_se0_b, enc_se1_w, enc_se1_b, enc_se2_w, enc_se2_b,
           enc_se3_w, enc_se3_b, enc_sc01_w, enc_sc01_b, enc_sc12_w,
           enc_sc12_b, enc_t001_w, enc_t001_b, enc_t012_w, enc_t012_b,
           enc_t101_w, enc_t101_b, enc_t112_w, enc_t112_b, enc_t201_w,
           enc_t201_b, enc_t212_w, enc_t212_b, emb21_w, emb21_b, emb11_w,
           emb11_b, emb10_w, emb10_b, fusion_w, fusion_b, memory_w0,
           memory_w1, memory_w2, memory_x):
    B, L = memory_x.shape[0], memory_x.shape[1]
    T = L - 1
    H = memory_x.shape[3]
    N = B * T

    x = memory_x[:, :, 0]
    d = (x[:, 1:] - x[:, :-1]).reshape(N, H, H)
    dp = jnp.pad(d.astype(jnp.bfloat16), ((0, 0), (1, 1), (1, 1)))
    cols = jnp.stack([dp[:, kh:kh + H:2, kw:kw + H:2]
                      for kh in range(3) for kw in range(3)], axis=-1)
    h0 = H // 2
    w0 = enc_se0_w.reshape(64, 9).T.astype(jnp.bfloat16)
    xp1 = pl.pallas_call(
        functools.partial(_se0_body, oh=h0),
        out_shape=jax.ShapeDtypeStruct((N, h0 + 2, h0 + 2, 64), jnp.bfloat16),
        grid=(N,),
        in_specs=[
            pl.BlockSpec((h0 * h0, 9), lambda i: (i, 0)),
            pl.BlockSpec((9, 64), lambda i: (0, 0)),
            pl.BlockSpec((1, 64), lambda i: (0, 0)),
        ],
        out_specs=pl.BlockSpec((1, h0 + 2, h0 + 2, 64),
                               lambda i: (i, 0, 0, 0)),
        compiler_params=pltpu.CompilerParams(
            dimension_semantics=("parallel",), vmem_limit_bytes=_VMEM),
    )(cols.reshape(N * h0 * h0, 9), w0,
      enc_se0_b.reshape(1, 64).astype(jnp.float32))

    (xph2,) = _conv3x3(xp1, "pad", enc_se1_w, enc_se1_b, 1, "elu", ["phase"])
    (xp3,) = _conv3x3(xph2, "phase", enc_se2_w, enc_se2_b, 2, "elu", ["pad"])
    x0r, x0ph = _conv3x3(xp3, "pad", enc_se3_w, enc_se3_b, 1, "elu",
                         ["rows", "phase"])
    x1r, x1ph = _conv3x3(x0ph, "phase", enc_sc01_w, enc_sc01_b, 2, "relu",
                         ["rows", "phase"])
    (x2r,) = _conv3x3(x1ph, "phase", enc_sc12_w, enc_sc12_b, 2, "relu",
                      ["rows"])

    mn2, mr2 = _norm_rows(memory_w2), memory_w2.astype(jnp.bfloat16)
    mn1, mr1 = _norm_rows(memory_w1), memory_w1.astype(jnp.bfloat16)
    mn0, mr0 = _norm_rows(memory_w0), memory_w0.astype(jnp.bfloat16)

    def f32b(v):
        return v.reshape(1, -1).astype(jnp.float32)

    cpar = pltpu.CompilerParams(
        dimension_semantics=("parallel", "arbitrary"), vmem_limit_bytes=_VMEM)
    bh = B // 2

    y2 = pl.pallas_call(
        functools.partial(_head2_body, bh=bh, T=T),
        out_shape=jax.ShapeDtypeStruct((B * 64, 1024), jnp.bfloat16),
        grid=(2, T),
        in_specs=[
            pl.BlockSpec((bh, 1, 64, 512), lambda i, t: (i, t, 0, 0)),
            pl.BlockSpec((1024, 512), lambda i, t: (0, t)),
            _bspec((1, 1024)),
            _bspec((512, 1024)), _bspec((1, 512)),
            _bspec((256, 512)), _bspec((256, 512)),
            _bspec((2048, 1024)), _bspec((1, 256)),
        ],
        out_specs=pl.BlockSpec((bh * 64, 1024), lambda i, t: (i, 0)),
        scratch_shapes=[pltpu.VMEM((bh * 64, 1024), jnp.float32)],
        compiler_params=cpar,
    )(x2r.reshape(B, T, 64, 512),
      enc_t201_w.reshape(1024, 2048).astype(jnp.bfloat16),
      f32b(enc_t201_b),
      enc_t212_w.reshape(512, 1024).astype(jnp.bfloat16), f32b(enc_t212_b),
      mn2, mr2, _convT_weight(emb21_w), f32b(emb21_b))

    y10, y11 = pl.pallas_call(
        functools.partial(_head1_body, bh=bh, T=T),
        out_shape=(jax.ShapeDtypeStruct((B * 256, 512), jnp.bfloat16),
                   jax.ShapeDtypeStruct((B * 256, 512), jnp.bfloat16)),
        grid=(2, T),
        in_specs=[
            pl.BlockSpec((bh, 1, 256, 256), lambda i, t: (i, t, 0, 0)),
            pl.BlockSpec((bh * 64, 1024), lambda i, t: (i, 0)),
            pl.BlockSpec((512, 256), lambda i, t: (0, t)),
            _bspec((1, 512)),
            _bspec((256, 512)), _bspec((1, 256)),
            _bspec((512, 256)), _bspec((512, 256)),
            _bspec((1024, 512)), _bspec((1, 128)),
            _bspec((1024, 512)), _bspec((1, 128)),
        ],
        out_specs=(pl.BlockSpec((bh * 256, 512), lambda i, t: (i, 0)),
                   pl.BlockSpec((bh * 256, 512), lambda i, t: (i, 0))),
        scratch_shapes=[pltpu.VMEM((bh * 256, 512), jnp.float32)],
        compiler_params=cpar,
    )(x1r.reshape(B, T, 256, 256), y2,
      enc_t101_w.reshape(512, 1024).astype(jnp.bfloat16), f32b(enc_t101_b),
      enc_t112_w.reshape(256, 512).astype(jnp.bfloat16), f32b(enc_t112_b),
      mn1, mr1, _convT_weight(emb10_w), f32b(emb10_b),
      _convT_weight(emb11_w), f32b(emb11_b))

    bq = B // 4
    out = pl.pallas_call(
        functools.partial(_head0_body, bh=bq, T=T),
        out_shape=jax.ShapeDtypeStruct((B, 128, 32, 32), jnp.float32),
        grid=(4, T),
        in_specs=[
            pl.BlockSpec((bq, 1, 1024, 128), lambda i, t: (i, t, 0, 0)),
            pl.BlockSpec((bq, 256, 512), lambda i, t: (i, 0, 0)),
            pl.BlockSpec((bq, 256, 512), lambda i, t: (i, 0, 0)),
            pl.BlockSpec((256, 128), lambda i, t: (0, t)),
            pl.BlockSpec((1, 256), lambda i, t: (0, 0)),
            pl.BlockSpec((128, 256), lambda i, t: (0, 0)),
            pl.BlockSpec((1, 128), lambda i, t: (0, 0)),
            pl.BlockSpec((1024, 128), lambda i, t: (0, 0)),
            pl.BlockSpec((1024, 128), lambda i, t: (0, 0)),
            pl.BlockSpec((384, 128), lambda i, t: (0, 0)),
            pl.BlockSpec((1, 128), lambda i, t: (0, 0)),
        ],
        out_specs=pl.BlockSpec((bq, 128, 32, 32), lambda i, t: (i, 0, 0, 0)),
        scratch_shapes=[pltpu.VMEM((bq * 1024, 256), jnp.float32)],
        compiler_params=cpar,
    )(x0r.reshape(B, T, 1024, 128),
      y10.reshape(B, 256, 512), y11.reshape(B, 256, 512),
      enc_t001_w.reshape(256, 512).astype(jnp.bfloat16), f32b(enc_t001_b),
      enc_t012_w.reshape(128, 256).astype(jnp.bfloat16), f32b(enc_t012_b),
      mn0, mr0, fusion_w.reshape(384, 128).astype(jnp.bfloat16),
      f32b(fusion_b))
    return out
